# baseline (speedup 1.0000x reference)
import functools

import jax
import jax.numpy as jnp
import numpy as np
from jax import lax
from jax.experimental import pallas as pl
from jax.experimental.pallas import tpu as pltpu
from jax.experimental.pallas import tpu_sc as plsc

N = 10000
E = 320000
D_IN = 128
D_H = 128
D_OUT = 64

NC = 2
NS = 16
NW = NC * NS

K = 128
NBUF = 4
CH = 160
EP = NS * K * CH
NPAD = 10240
RPT = NPAD // NS

RB = 1000
NB = N // RB


def _sc_mesh():
    return plsc.VectorSubcoreMesh(
        core_axis_name="c", subcore_axis_name="s", num_cores=NC, num_subcores=NS)


@functools.cache
def _make_sc_deg():
    @functools.partial(
        pl.kernel,
        out_type=jax.ShapeDtypeStruct((NC, NPAD), jnp.float32),
        mesh=_sc_mesh(),
        scratch_types=[
            pltpu.VMEM((CH, K), jnp.int32),
            pltpu.VMEM((K,), jnp.float32),
            pltpu.VMEM((RPT,), jnp.float32),
            pltpu.VMEM_SHARED((NPAD,), jnp.float32),
            pltpu.SemaphoreType.DMA,
        ],
    )
    def _sc_deg(dst_hbm, out_hbm, dst_all, ones_v, zbuf, acc, sem):
        cid = lax.axis_index("c")
        sid = lax.axis_index("s")
        w = cid * NS + sid

        zeros16 = jnp.zeros((16,), jnp.float32)
        ones16 = jnp.ones((16,), jnp.float32)

        @pl.loop(0, K // 16)
        def _fill(i):
            ones_v[pl.ds(i * 16, 16)] = ones16

        @pl.loop(0, RPT // 16)
        def _zb(i):
            zbuf[pl.ds(i * 16, 16)] = zeros16

        pltpu.sync_copy(dst_hbm.at[pl.ds(w * CH, CH)], dst_all)
        pltpu.sync_copy(zbuf, acc.at[pl.ds(sid * RPT, RPT)])
        plsc.subcore_barrier()

        @pl.loop(0, CH // 8)
        def _edges(g):
            descs = [
                pltpu.async_copy(ones_v, acc.at[dst_all.at[g * 8 + j]], sem,
                                 add=True)
                for j in range(8)
            ]
            for dsc in descs:
                dsc.wait()

        plsc.subcore_barrier()
        pltpu.sync_copy(acc.at[pl.ds(sid * RPT, RPT)],
                        out_hbm.at[cid, pl.ds(sid * RPT, RPT)])

    return _sc_deg


@functools.cache
def _make_sc_agg(d):
    @functools.partial(
        pl.kernel,
        out_type=jax.ShapeDtypeStruct((NC, NPAD, d), jnp.float32),
        mesh=_sc_mesh(),
        scratch_types=[
            pltpu.VMEM((CH, K), jnp.int32),
            pltpu.VMEM((CH, K), jnp.int32),
            [pltpu.VMEM((K, d), jnp.float32) for _ in range(NBUF)],
            pltpu.VMEM_SHARED((NPAD, d), jnp.float32),
            [pltpu.SemaphoreType.DMA for _ in range(NBUF)],
        ],
        compiler_params=pltpu.CompilerParams(use_tc_tiling_on_sc=False),
    )
    def agg(h_hbm, src_hbm, dst_hbm, out_hbm, src_all, dst_all, rows, acc,
            sem_g):
        cid = lax.axis_index("c")
        sid = lax.axis_index("s")
        w = cid * NS + sid

        zeros16 = jnp.zeros((16,), jnp.float32)

        @pl.loop(0, K)
        def _zr(i):
            for j in range(d // 16):
                rows[0][i, pl.ds(j * 16, 16)] = zeros16

        pltpu.sync_copy(src_hbm.at[pl.ds(w * CH, CH)], src_all)
        pltpu.sync_copy(dst_hbm.at[pl.ds(w * CH, CH)], dst_all)
        for t in range(RPT // K):
            pltpu.sync_copy(rows[0], acc.at[pl.ds(sid * RPT + t * K, K)])
        plsc.subcore_barrier()

        for b in range(NBUF):
            pltpu.async_copy(h_hbm.at[src_all.at[b]], rows[b], sem_g[b])

        @pl.loop(0, CH // NBUF)
        def _edges(i):
            for b in range(NBUF):
                t = i * NBUF + b
                pltpu.make_async_copy(
                    h_hbm.at[pl.ds(0, K)], rows[b], sem_g[b]).wait()
                pltpu.sync_copy(rows[b], acc.at[dst_all.at[t]], add=True)

                @pl.when(t + NBUF < CH)
                def _refill():
                    pltpu.async_copy(
                        h_hbm.at[src_all.at[t + NBUF]], rows[b], sem_g[b])

        plsc.subcore_barrier()
        pltpu.sync_copy(acc.at[pl.ds(sid * RPT, RPT)],
                        out_hbm.at[cid, pl.ds(sid * RPT, RPT)])

    return agg


def _tc1_body(dp_ref, x_ref, w1_ref, h1_ref, dis_ref):
    deg = 0.5 * (dp_ref[0, 0] + dp_ref[0, 1]) + 1.0
    dis = lax.rsqrt(deg)
    h = jnp.dot(x_ref[...], w1_ref[...], preferred_element_type=jnp.float32)
    h1_ref[...] = dis[:, None] * h
    dis_ref[0, 0] = dis


def _tc1(degp3, x, W1):
    return pl.pallas_call(
        _tc1_body,
        grid=(NB,),
        in_specs=[
            pl.BlockSpec((1, NC, RB), lambda i: (i, 0, 0)),
            pl.BlockSpec((RB, D_IN), lambda i: (i, 0)),
            pl.BlockSpec((D_IN, D_H), lambda i: (0, 0)),
        ],
        out_specs=[
            pl.BlockSpec((RB, D_H), lambda i: (i, 0)),
            pl.BlockSpec((1, 1, RB), lambda i: (i, 0, 0)),
        ],
        out_shape=[
            jax.ShapeDtypeStruct((N, D_H), jnp.float32),
            jax.ShapeDtypeStruct((NB, 1, RB), jnp.float32),
        ],
    )(degp3, x, W1)


def _tc2_body(p_ref, h1_ref, dis_ref, w2_ref, b1_ref, out_ref):
    dis = dis_ref[0, 0]
    s = jnp.concatenate([p_ref[0], p_ref[1]], axis=1) + h1_ref[...]
    h = jnp.maximum(dis[:, None] * s + b1_ref[...][None, :], 0.0)
    out_ref[...] = dis[:, None] * jnp.dot(h, w2_ref[...],
                                          preferred_element_type=jnp.float32)


def _tc2(parts1, hh1, dis2, W2, b1):
    return pl.pallas_call(
        _tc2_body,
        grid=(NB,),
        in_specs=[
            pl.BlockSpec((2, RB, D_H // 2), lambda i: (0, i, 0)),
            pl.BlockSpec((RB, D_H), lambda i: (i, 0)),
            pl.BlockSpec((1, 1, RB), lambda i: (i, 0, 0)),
            pl.BlockSpec((D_H, D_OUT), lambda i: (0, 0)),
            pl.BlockSpec((D_H,), lambda i: (0,)),
        ],
        out_specs=pl.BlockSpec((RB, D_OUT), lambda i: (i, 0)),
        out_shape=jax.ShapeDtypeStruct((N, D_OUT), jnp.float32),
    )(parts1, hh1, dis2, W2, b1)


def _tc3_body(p_ref, h2_ref, dis_ref, b2_ref, out_ref):
    dis = dis_ref[0, 0]
    s = jnp.concatenate([p_ref[0], p_ref[1]], axis=1) + h2_ref[...]
    out_ref[...] = dis[:, None] * s + b2_ref[...][None, :]


def _tc3(parts2, hh2, dis2, b2):
    return pl.pallas_call(
        _tc3_body,
        grid=(NB,),
        in_specs=[
            pl.BlockSpec((2, RB, D_OUT // 2), lambda i: (0, i, 0)),
            pl.BlockSpec((RB, D_OUT), lambda i: (i, 0)),
            pl.BlockSpec((1, 1, RB), lambda i: (i, 0, 0)),
            pl.BlockSpec((D_OUT,), lambda i: (0,)),
        ],
        out_specs=pl.BlockSpec((RB, D_OUT), lambda i: (i, 0)),
        out_shape=jax.ShapeDtypeStruct((N, D_OUT), jnp.float32),
    )(parts2, hh2, dis2, b2)


def kernel(x, edge_index, W1, b1, W2, b2):
    pad = EP - E
    ppt = pad // NS
    rpt_e = E // NS
    trash = jnp.asarray(
        (N + np.arange(pad, dtype=np.int32) % (NPAD - N)).reshape(NS, ppt))
    psrc = jnp.asarray((np.arange(pad, dtype=np.int32) % N).reshape(NS, ppt))
    src_t = jnp.concatenate(
        [edge_index[0].reshape(NS, rpt_e), psrc], axis=1).reshape(NS * CH, K)
    dst_t = jnp.concatenate(
        [edge_index[1].reshape(NS, rpt_e), trash], axis=1).reshape(NS * CH, K)
    src_p = jnp.concatenate([2 * src_t, 2 * src_t + 1])
    dst_p = jnp.concatenate([dst_t, dst_t])

    degp = _make_sc_deg()(dst_p)
    degp3 = degp[:, :N].reshape(NC, NB, RB).transpose(1, 0, 2)

    h1, dis2 = _tc1(degp3, x, W1)
    hh1 = h1.reshape(2 * N, D_H // 2)
    parts1 = _make_sc_agg(D_H // 2)(hh1, src_p, dst_p)
    h2 = _tc2(parts1, h1, dis2, W2, b1)
    hh2 = h2.reshape(2 * N, D_OUT // 2)
    parts2 = _make_sc_agg(D_OUT // 2)(hh2, src_p, dst_p)
    return _tc3(parts2, h2, dis2, b2)


# speedup vs baseline: 1.0242x; 1.0242x over previous
import functools

import jax
import jax.numpy as jnp
import numpy as np
from jax import lax
from jax.experimental import pallas as pl
from jax.experimental.pallas import tpu as pltpu
from jax.experimental.pallas import tpu_sc as plsc

N = 10000
E = 320000
D_IN = 128
D_H = 128
D_OUT = 64

NC = 2
NS = 16
NW = NC * NS

K = 128
NBUF = 4
CH = 160
EP = NS * K * CH
NPAD = 10240
RPT = NPAD // NS

RB = 1000
NB = N // RB


def _sc_mesh():
    return plsc.VectorSubcoreMesh(
        core_axis_name="c", subcore_axis_name="s", num_cores=NC, num_subcores=NS)


@functools.cache
def _make_sc_deg():
    @functools.partial(
        pl.kernel,
        out_type=jax.ShapeDtypeStruct((NC, NPAD), jnp.float32),
        mesh=_sc_mesh(),
        scratch_types=[
            pltpu.VMEM((CH, K), jnp.int32),
            pltpu.VMEM((K,), jnp.float32),
            pltpu.VMEM((RPT,), jnp.float32),
            pltpu.VMEM_SHARED((NPAD,), jnp.float32),
            pltpu.SemaphoreType.DMA,
        ],
    )
    def _sc_deg(dst_hbm, out_hbm, dst_all, ones_v, zbuf, acc, sem):
        cid = lax.axis_index("c")
        sid = lax.axis_index("s")
        w = cid * NS + sid

        zeros16 = jnp.zeros((16,), jnp.float32)
        ones16 = jnp.ones((16,), jnp.float32)

        @pl.loop(0, K // 16)
        def _fill(i):
            ones_v[pl.ds(i * 16, 16)] = ones16

        @pl.loop(0, RPT // 16)
        def _zb(i):
            zbuf[pl.ds(i * 16, 16)] = zeros16

        pltpu.sync_copy(dst_hbm.at[pl.ds(w * CH, CH)], dst_all)
        pltpu.sync_copy(zbuf, acc.at[pl.ds(sid * RPT, RPT)])
        plsc.subcore_barrier()

        @pl.loop(0, CH // 8)
        def _edges(g):
            descs = [
                pltpu.async_copy(ones_v, acc.at[dst_all.at[g * 8 + j]], sem,
                                 add=True)
                for j in range(8)
            ]
            for dsc in descs:
                dsc.wait()

        plsc.subcore_barrier()
        pltpu.sync_copy(acc.at[pl.ds(sid * RPT, RPT)],
                        out_hbm.at[cid, pl.ds(sid * RPT, RPT)])

    return _sc_deg


@functools.cache
def _make_sc_agg(d, nbuf=NBUF):
    @functools.partial(
        pl.kernel,
        out_type=jax.ShapeDtypeStruct((NC, NPAD, d), jnp.float32),
        mesh=_sc_mesh(),
        scratch_types=[
            pltpu.VMEM((CH, K), jnp.int32),
            pltpu.VMEM((CH, K), jnp.int32),
            [pltpu.VMEM((K, d), jnp.float32) for _ in range(nbuf)],
            pltpu.VMEM_SHARED((NPAD, d), jnp.float32),
            [pltpu.SemaphoreType.DMA for _ in range(nbuf)],
        ],
        compiler_params=pltpu.CompilerParams(use_tc_tiling_on_sc=False),
    )
    def agg(h_hbm, src_hbm, dst_hbm, out_hbm, src_all, dst_all, rows, acc,
            sem_g):
        cid = lax.axis_index("c")
        sid = lax.axis_index("s")
        w = cid * NS + sid

        zeros16 = jnp.zeros((16,), jnp.float32)

        @pl.loop(0, K)
        def _zr(i):
            for j in range(d // 16):
                rows[0][i, pl.ds(j * 16, 16)] = zeros16

        pltpu.sync_copy(src_hbm.at[pl.ds(w * CH, CH)], src_all)
        pltpu.sync_copy(dst_hbm.at[pl.ds(w * CH, CH)], dst_all)
        for t in range(RPT // K):
            pltpu.sync_copy(rows[0], acc.at[pl.ds(sid * RPT + t * K, K)])
        plsc.subcore_barrier()

        for b in range(nbuf):
            pltpu.async_copy(h_hbm.at[src_all.at[b]], rows[b], sem_g[b])

        @pl.loop(0, CH // nbuf)
        def _edges(i):
            for b in range(nbuf):
                t = i * nbuf + b
                pltpu.make_async_copy(
                    h_hbm.at[pl.ds(0, K)], rows[b], sem_g[b]).wait()
                pltpu.sync_copy(rows[b], acc.at[dst_all.at[t]], add=True)

                @pl.when(t + nbuf < CH)
                def _refill():
                    pltpu.async_copy(
                        h_hbm.at[src_all.at[t + nbuf]], rows[b], sem_g[b])

        plsc.subcore_barrier()
        pltpu.sync_copy(acc.at[pl.ds(sid * RPT, RPT)],
                        out_hbm.at[cid, pl.ds(sid * RPT, RPT)])

    return agg


def _tc1_body(dp_ref, x_ref, w1_ref, h1_ref, dis_ref):
    deg = 0.5 * (dp_ref[0, 0] + dp_ref[0, 1]) + 1.0
    dis = lax.rsqrt(deg)
    h = jnp.dot(x_ref[...], w1_ref[...], preferred_element_type=jnp.float32)
    h1_ref[...] = dis[:, None] * h
    dis_ref[0, 0] = dis


def _tc1(degp3, x, W1):
    return pl.pallas_call(
        _tc1_body,
        grid=(NB,),
        in_specs=[
            pl.BlockSpec((1, NC, RB), lambda i: (i, 0, 0)),
            pl.BlockSpec((RB, D_IN), lambda i: (i, 0)),
            pl.BlockSpec((D_IN, D_H), lambda i: (0, 0)),
        ],
        out_specs=[
            pl.BlockSpec((RB, D_H), lambda i: (i, 0)),
            pl.BlockSpec((1, 1, RB), lambda i: (i, 0, 0)),
        ],
        out_shape=[
            jax.ShapeDtypeStruct((N, D_H), jnp.float32),
            jax.ShapeDtypeStruct((NB, 1, RB), jnp.float32),
        ],
    )(degp3, x, W1)


def _tc2_body(p_ref, h1_ref, dis_ref, w2_ref, b1_ref, out_ref):
    dis = dis_ref[0, 0]
    s = jnp.concatenate([p_ref[0], p_ref[1]], axis=1) + h1_ref[...]
    h = jnp.maximum(dis[:, None] * s + b1_ref[...][None, :], 0.0)
    out_ref[...] = dis[:, None] * jnp.dot(h, w2_ref[...],
                                          preferred_element_type=jnp.float32)


def _tc2(parts1, hh1, dis2, W2, b1):
    return pl.pallas_call(
        _tc2_body,
        grid=(NB,),
        in_specs=[
            pl.BlockSpec((2, RB, D_H // 2), lambda i: (0, i, 0)),
            pl.BlockSpec((RB, D_H), lambda i: (i, 0)),
            pl.BlockSpec((1, 1, RB), lambda i: (i, 0, 0)),
            pl.BlockSpec((D_H, D_OUT), lambda i: (0, 0)),
            pl.BlockSpec((D_H,), lambda i: (0,)),
        ],
        out_specs=pl.BlockSpec((RB, D_OUT), lambda i: (i, 0)),
        out_shape=jax.ShapeDtypeStruct((N, D_OUT), jnp.float32),
    )(parts1, hh1, dis2, W2, b1)


def _tc3_body(p_ref, h2_ref, dis_ref, b2_ref, out_ref):
    dis = dis_ref[0, 0]
    s = jnp.concatenate([p_ref[0], p_ref[1]], axis=1) + h2_ref[...]
    out_ref[...] = dis[:, None] * s + b2_ref[...][None, :]


def _tc3(parts2, hh2, dis2, b2):
    return pl.pallas_call(
        _tc3_body,
        grid=(NB,),
        in_specs=[
            pl.BlockSpec((2, RB, D_OUT // 2), lambda i: (0, i, 0)),
            pl.BlockSpec((RB, D_OUT), lambda i: (i, 0)),
            pl.BlockSpec((1, 1, RB), lambda i: (i, 0, 0)),
            pl.BlockSpec((D_OUT,), lambda i: (0,)),
        ],
        out_specs=pl.BlockSpec((RB, D_OUT), lambda i: (i, 0)),
        out_shape=jax.ShapeDtypeStruct((N, D_OUT), jnp.float32),
    )(parts2, hh2, dis2, b2)


def kernel(x, edge_index, W1, b1, W2, b2):
    pad = EP - E
    ppt = pad // NS
    rpt_e = E // NS
    trash = jnp.asarray(
        (N + np.arange(pad, dtype=np.int32) % (NPAD - N)).reshape(NS, ppt))
    psrc = jnp.asarray((np.arange(pad, dtype=np.int32) % N).reshape(NS, ppt))
    src_t = jnp.concatenate(
        [edge_index[0].reshape(NS, rpt_e), psrc], axis=1).reshape(NS * CH, K)
    dst_t = jnp.concatenate(
        [edge_index[1].reshape(NS, rpt_e), trash], axis=1).reshape(NS * CH, K)
    src_p = jnp.concatenate([2 * src_t, 2 * src_t + 1])
    dst_p = jnp.concatenate([dst_t, dst_t])

    degp = _make_sc_deg()(dst_p)
    degp3 = degp[:, :N].reshape(NC, NB, RB).transpose(1, 0, 2)

    h1, dis2 = _tc1(degp3, x, W1)
    hh1 = h1.reshape(2 * N, D_H // 2)
    parts1 = _make_sc_agg(D_H // 2)(hh1, src_p, dst_p)
    h2 = _tc2(parts1, h1, dis2, W2, b1)
    hh2 = h2.reshape(2 * N, D_OUT // 2)
    parts2 = _make_sc_agg(D_OUT // 2, 8)(hh2, src_p, dst_p)
    return _tc3(parts2, h2, dis2, b2)


# speedup vs baseline: 1.0659x; 1.0407x over previous
import functools

import jax
import jax.numpy as jnp
import numpy as np
from jax import lax
from jax.experimental import pallas as pl
from jax.experimental.pallas import tpu as pltpu
from jax.experimental.pallas import tpu_sc as plsc

N = 10000
E = 320000
D_IN = 128
D_H = 128
D_OUT = 64

NC = 2
NS = 16
NW = NC * NS

K = 128
NBUF = 4
CH = 160
EP = NS * K * CH
NPAD = 10240
RPT = NPAD // NS

RB = 1000
NB = N // RB


def _sc_mesh():
    return plsc.VectorSubcoreMesh(
        core_axis_name="c", subcore_axis_name="s", num_cores=NC, num_subcores=NS)


@functools.cache
def _make_sc_deg():
    @functools.partial(
        pl.kernel,
        out_type=jax.ShapeDtypeStruct((NC, NPAD), jnp.float32),
        mesh=_sc_mesh(),
        scratch_types=[
            pltpu.VMEM((CH, K), jnp.int32),
            pltpu.VMEM((K,), jnp.float32),
            pltpu.VMEM((RPT,), jnp.float32),
            pltpu.VMEM_SHARED((NPAD,), jnp.float32),
            pltpu.SemaphoreType.DMA,
        ],
    )
    def _sc_deg(dst_hbm, out_hbm, dst_all, ones_v, zbuf, acc, sem):
        cid = lax.axis_index("c")
        sid = lax.axis_index("s")
        w = cid * NS + sid

        zeros16 = jnp.zeros((16,), jnp.float32)
        ones16 = jnp.ones((16,), jnp.float32)

        @pl.loop(0, K // 16)
        def _fill(i):
            ones_v[pl.ds(i * 16, 16)] = ones16

        @pl.loop(0, RPT // 16)
        def _zb(i):
            zbuf[pl.ds(i * 16, 16)] = zeros16

        pltpu.sync_copy(dst_hbm.at[pl.ds(w * CH, CH)], dst_all)
        pltpu.sync_copy(zbuf, acc.at[pl.ds(sid * RPT, RPT)])
        plsc.subcore_barrier()

        @pl.loop(0, CH // 8)
        def _edges(g):
            descs = [
                pltpu.async_copy(ones_v, acc.at[dst_all.at[g * 8 + j]], sem,
                                 add=True)
                for j in range(8)
            ]
            for dsc in descs:
                dsc.wait()

        plsc.subcore_barrier()
        pltpu.sync_copy(acc.at[pl.ds(sid * RPT, RPT)],
                        out_hbm.at[cid, pl.ds(sid * RPT, RPT)])

    return _sc_deg


@functools.cache
def _make_sc_agg(d, nbuf=NBUF, merged_out=False):
    out_t = (jax.ShapeDtypeStruct((NPAD, 2 * d), jnp.float32) if merged_out
             else jax.ShapeDtypeStruct((NC, NPAD, d), jnp.float32))
    @functools.partial(
        pl.kernel,
        out_type=out_t,
        mesh=_sc_mesh(),
        scratch_types=[
            pltpu.VMEM((CH, K), jnp.int32),
            pltpu.VMEM((CH, K), jnp.int32),
            [pltpu.VMEM((K, d), jnp.float32) for _ in range(nbuf)],
            pltpu.VMEM_SHARED((NPAD, d), jnp.float32),
            [pltpu.SemaphoreType.DMA for _ in range(nbuf)],
        ],
        compiler_params=pltpu.CompilerParams(use_tc_tiling_on_sc=False),
    )
    def agg(h_hbm, src_hbm, dst_hbm, out_hbm, src_all, dst_all, rows, acc,
            sem_g):
        cid = lax.axis_index("c")
        sid = lax.axis_index("s")
        w = cid * NS + sid

        zeros16 = jnp.zeros((16,), jnp.float32)

        @pl.loop(0, K)
        def _zr(i):
            for j in range(d // 16):
                rows[0][i, pl.ds(j * 16, 16)] = zeros16

        pltpu.sync_copy(src_hbm.at[pl.ds(w * CH, CH)], src_all)
        pltpu.sync_copy(dst_hbm.at[pl.ds(w * CH, CH)], dst_all)
        for t in range(RPT // K):
            pltpu.sync_copy(rows[0], acc.at[pl.ds(sid * RPT + t * K, K)])
        plsc.subcore_barrier()

        for b in range(nbuf):
            pltpu.async_copy(h_hbm.at[src_all.at[b]], rows[b], sem_g[b])

        @pl.loop(0, CH // nbuf)
        def _edges(i):
            for b in range(nbuf):
                t = i * nbuf + b
                pltpu.make_async_copy(
                    h_hbm.at[pl.ds(0, K)], rows[b], sem_g[b]).wait()
                pltpu.sync_copy(rows[b], acc.at[dst_all.at[t]], add=True)

                @pl.when(t + nbuf < CH)
                def _refill():
                    pltpu.async_copy(
                        h_hbm.at[src_all.at[t + nbuf]], rows[b], sem_g[b])

        plsc.subcore_barrier()
        if merged_out:
            pltpu.sync_copy(acc.at[pl.ds(sid * RPT, RPT)],
                            out_hbm.at[pl.ds(sid * RPT, RPT),
                                       pl.ds(cid * d, d)])
        else:
            pltpu.sync_copy(acc.at[pl.ds(sid * RPT, RPT)],
                            out_hbm.at[cid, pl.ds(sid * RPT, RPT)])

    return agg


def _tc1_body(dp_ref, x_ref, w1_ref, h1_ref, dis_ref):
    deg = 0.5 * (dp_ref[0, 0] + dp_ref[0, 1]) + 1.0
    dis = lax.rsqrt(deg)
    h = jnp.dot(x_ref[...], w1_ref[...], preferred_element_type=jnp.float32)
    h1_ref[...] = dis[:, None] * h
    dis_ref[0, 0] = dis


def _tc1(degp3, x, W1):
    return pl.pallas_call(
        _tc1_body,
        grid=(NB,),
        in_specs=[
            pl.BlockSpec((1, NC, RB), lambda i: (i, 0, 0)),
            pl.BlockSpec((RB, D_IN), lambda i: (i, 0)),
            pl.BlockSpec((D_IN, D_H), lambda i: (0, 0)),
        ],
        out_specs=[
            pl.BlockSpec((RB, D_H), lambda i: (i, 0)),
            pl.BlockSpec((1, 1, RB), lambda i: (i, 0, 0)),
        ],
        out_shape=[
            jax.ShapeDtypeStruct((N, D_H), jnp.float32),
            jax.ShapeDtypeStruct((NB, 1, RB), jnp.float32),
        ],
    )(degp3, x, W1)


def _tc2_body(p_ref, h1_ref, dis_ref, w2_ref, b1_ref, out_ref):
    dis = dis_ref[0, 0]
    s = p_ref[...] + h1_ref[...]
    h = jnp.maximum(dis[:, None] * s + b1_ref[...][None, :], 0.0)
    out_ref[...] = dis[:, None] * jnp.dot(h, w2_ref[...],
                                          preferred_element_type=jnp.float32)


def _tc2(parts1, hh1, dis2, W2, b1):
    return pl.pallas_call(
        _tc2_body,
        grid=(NB,),
        in_specs=[
            pl.BlockSpec((RB, D_H), lambda i: (i, 0)),
            pl.BlockSpec((RB, D_H), lambda i: (i, 0)),
            pl.BlockSpec((1, 1, RB), lambda i: (i, 0, 0)),
            pl.BlockSpec((D_H, D_OUT), lambda i: (0, 0)),
            pl.BlockSpec((D_H,), lambda i: (0,)),
        ],
        out_specs=pl.BlockSpec((RB, D_OUT), lambda i: (i, 0)),
        out_shape=jax.ShapeDtypeStruct((N, D_OUT), jnp.float32),
    )(parts1, hh1, dis2, W2, b1)


def _tc3_body(p_ref, h2_ref, dis_ref, b2_ref, out_ref):
    dis = dis_ref[0, 0]
    s = jnp.concatenate([p_ref[0], p_ref[1]], axis=1) + h2_ref[...]
    out_ref[...] = dis[:, None] * s + b2_ref[...][None, :]


def _tc3(parts2, hh2, dis2, b2):
    return pl.pallas_call(
        _tc3_body,
        grid=(NB,),
        in_specs=[
            pl.BlockSpec((2, RB, D_OUT // 2), lambda i: (0, i, 0)),
            pl.BlockSpec((RB, D_OUT), lambda i: (i, 0)),
            pl.BlockSpec((1, 1, RB), lambda i: (i, 0, 0)),
            pl.BlockSpec((D_OUT,), lambda i: (0,)),
        ],
        out_specs=pl.BlockSpec((RB, D_OUT), lambda i: (i, 0)),
        out_shape=jax.ShapeDtypeStruct((N, D_OUT), jnp.float32),
    )(parts2, hh2, dis2, b2)


def kernel(x, edge_index, W1, b1, W2, b2):
    pad = EP - E
    ppt = pad // NS
    rpt_e = E // NS
    trash = jnp.asarray(
        (N + np.arange(pad, dtype=np.int32) % (NPAD - N)).reshape(NS, ppt))
    psrc = jnp.asarray((np.arange(pad, dtype=np.int32) % N).reshape(NS, ppt))
    src_t = jnp.concatenate(
        [edge_index[0].reshape(NS, rpt_e), psrc], axis=1).reshape(NS * CH, K)
    dst_t = jnp.concatenate(
        [edge_index[1].reshape(NS, rpt_e), trash], axis=1).reshape(NS * CH, K)
    src_p = jnp.concatenate([2 * src_t, 2 * src_t + 1])
    dst_p = jnp.concatenate([dst_t, dst_t])

    degp = _make_sc_deg()(dst_p)
    degp3 = degp[:, :N].reshape(NC, NB, RB).transpose(1, 0, 2)

    h1, dis2 = _tc1(degp3, x, W1)
    hh1 = h1.reshape(2 * N, D_H // 2)
    parts1 = _make_sc_agg(D_H // 2, NBUF, True)(hh1, src_p, dst_p)
    h2 = _tc2(parts1, h1, dis2, W2, b1)
    hh2 = h2.reshape(2 * N, D_OUT // 2)
    parts2 = _make_sc_agg(D_OUT // 2, 8)(hh2, src_p, dst_p)
    return _tc3(parts2, h2, dis2, b2)
